# baseline (speedup 1.0000x reference)
import functools

import jax
import jax.numpy as jnp
import numpy as np
from jax import lax
from jax.experimental import pallas as pl
from jax.experimental.pallas import tpu as pltpu
from jax.experimental.pallas import tpu_sc as plsc

_B, _C, _T, _H, _W = 8, 192, 32, 28, 28
_D = _H * _W
_G = _B * _C
_NW = 32

_PERM = (31, 7, 4, 29, 16, 19, 2, 5, 30, 3, 22, 6, 18, 10, 11, 15, 20, 8,
         24, 9, 25, 13, 14, 17, 23, 0, 21, 26, 1, 28, 27, 12)


@functools.partial(
    pl.kernel,
    mesh=plsc.VectorSubcoreMesh(core_axis_name="c", subcore_axis_name="s"),
    out_type=jax.ShapeDtypeStruct((_G, _T, _D), jnp.float32),
    scratch_types=[pltpu.SemaphoreType.DMA],
    compiler_params=pltpu.CompilerParams(use_tc_tiling_on_sc=False),
)
def _shuffle_t(x_hbm, out_hbm, sem):
    wid = lax.axis_index("s") * 2 + lax.axis_index("c")
    for t in range(_T):
        @pl.when(wid == t)
        def _copy(t=t):
            pltpu.async_copy(
                x_hbm.at[:, _PERM[t], :], out_hbm.at[:, t, :], sem
            ).wait()


def kernel(x):
    x3 = x.reshape(_G, _T, _D)
    out3 = _shuffle_t(x3)
    return out3.reshape(_B, _C, _T, _H, _W)


# speedup vs baseline: 2.0938x; 2.0938x over previous
import functools

import jax
import jax.numpy as jnp
from jax import lax
from jax.experimental import pallas as pl
from jax.experimental.pallas import tpu as pltpu
from jax.experimental.pallas import tpu_sc as plsc

_B, _C, _T, _H, _W = 8, 192, 32, 28, 28
_D = _H * _W
_R = _B * _C * _T
_NW = 32
_RPW = _R // _NW
_CH = 64
_NCH = _RPW // _CH


@functools.partial(
    pl.kernel,
    mesh=plsc.VectorSubcoreMesh(core_axis_name="c", subcore_axis_name="s"),
    out_type=jax.ShapeDtypeStruct((_R, _D), jnp.float32),
    scratch_types=[
        pltpu.VMEM((_RPW,), jnp.int32),
        pltpu.VMEM((_CH, _D), jnp.float32),
        pltpu.VMEM((_CH, _D), jnp.float32),
        pltpu.SemaphoreType.DMA,
        pltpu.SemaphoreType.DMA,
        pltpu.SemaphoreType.DMA,
        pltpu.SemaphoreType.DMA,
    ],
    compiler_params=pltpu.CompilerParams(use_tc_tiling_on_sc=False),
)
def _shuffle_rows(x_hbm, idx_hbm, out_hbm, idx_v, buf0, buf1, g0, g1, s0, s1):
    wid = lax.axis_index("s") * 2 + lax.axis_index("c")
    base = wid * _RPW
    pltpu.sync_copy(idx_hbm.at[pl.ds(base, _RPW)], idx_v)
    bufs = (buf0, buf1)
    gsem = (g0, g1)
    ssem = (s0, s1)

    def gather(c):
        return pltpu.async_copy(
            x_hbm.at[idx_v.at[pl.ds(c * _CH, _CH)]], bufs[c % 2], gsem[c % 2]
        )

    def store(c):
        return pltpu.async_copy(
            bufs[c % 2], out_hbm.at[pl.ds(base + c * _CH, _CH)], ssem[c % 2]
        )

    gathers = [None] * _NCH
    gathers[0] = gather(0)
    if _NCH > 1:
        gathers[1] = gather(1)
    for c in range(_NCH):
        gathers[c].wait()
        st = store(c)
        st.wait()
        if c + 2 < _NCH:
            gathers[c + 2] = gather(c + 2)


def kernel(x):
    perm = jnp.array(
        (31, 7, 4, 29, 16, 19, 2, 5, 30, 3, 22, 6, 18, 10, 11, 15, 20, 8,
         24, 9, 25, 13, 14, 17, 23, 0, 21, 26, 1, 28, 27, 12),
        dtype=jnp.int32,
    )
    row_idx = (
        jnp.arange(_R // _T, dtype=jnp.int32)[:, None] * _T + perm[None, :]
    ).reshape(_R)
    x2d = x.reshape(_R, _D)
    out2d = _shuffle_rows(x2d, row_idx)
    return out2d.reshape(_B, _C, _T, _H, _W)


# speedup vs baseline: 43.6689x; 20.8561x over previous
import jax
import jax.numpy as jnp
from jax.experimental import pallas as pl

_B, _C, _T, _H, _W = 8, 192, 32, 28, 28
_PERM = (31, 7, 4, 29, 16, 19, 2, 5, 30, 3, 22, 6, 18, 10, 11, 15, 20, 8,
         24, 9, 25, 13, 14, 17, 23, 0, 21, 26, 1, 28, 27, 12)
_BH = 4


def _body(x_ref, o_ref):
    for t in range(_T):
        o_ref[:, :, :, t, :] = x_ref[:, :, :, _PERM[t], :]


def kernel(x):
    xt = jnp.transpose(x, (0, 3, 4, 2, 1))
    out_t = pl.pallas_call(
        _body,
        grid=(_B, _H // _BH),
        in_specs=[
            pl.BlockSpec(
                (1, _BH, _W, _T, _C), lambda i, j: (i, j, 0, 0, 0)
            )
        ],
        out_specs=pl.BlockSpec(
            (1, _BH, _W, _T, _C), lambda i, j: (i, j, 0, 0, 0)
        ),
        out_shape=jax.ShapeDtypeStruct((_B, _H, _W, _T, _C), jnp.float32),
    )(xt)
    return jnp.transpose(out_t, (0, 4, 3, 1, 2))
